# baseline (speedup 1.0000x reference)
import jax
import jax.numpy as jnp
from jax import lax
from jax.experimental import pallas as pl
from jax.experimental.pallas import tpu as pltpu
from jax.experimental.pallas import tpu_sc as plsc

N = 10000
E = 160000
D = 256

NC = 2
NS = 16
L = 16

NP = 10240
COLS = NP // NS
DEG_EPT = E // NS

PW = 4
NPASS = D // (NC * NS * PW)
EB = 8000
NCH = E // EB

BN = 1280


def _mesh():
    return plsc.VectorSubcoreMesh(core_axis_name="c", subcore_axis_name="s",
                                  num_cores=NC, num_subcores=NS)


def _deg_body(dst_hbm, dis_hbm, dstv, degv, colbuf, disv, spm, semr):
    c = lax.axis_index("c")
    s = lax.axis_index("s")
    zero16 = jnp.zeros((L,), jnp.float32)
    ones16 = jnp.ones((L,), jnp.float32)

    def zloop(i, _):
        degv[pl.ds(i * L, L)] = zero16
        return 0
    lax.fori_loop(0, NP // L, zloop, 0)

    pltpu.sync_copy(dst_hbm.at[pl.ds(s * DEG_EPT, DEG_EPT)], dstv)

    def hist(i, _):
        d = dstv[pl.ds(i * L, L)]
        plsc.addupdate_scatter(degv, [d], ones16)
        return 0
    lax.fori_loop(0, DEG_EPT // L, hist, 0)

    pltpu.sync_copy(degv, spm.at[s])
    plsc.subcore_barrier()

    for r in range(NS):
        pltpu.make_async_copy(spm.at[r, pl.ds(s * COLS, COLS)],
                              colbuf.at[r], semr).start()
    for r in range(NS):
        pltpu.make_async_copy(spm.at[r, pl.ds(s * COLS, COLS)],
                              colbuf.at[r], semr).wait()

    def red(k, _):
        acc = colbuf[0, pl.ds(k * L, L)]
        for r in range(1, NS):
            acc = acc + colbuf[r, pl.ds(k * L, L)]
        deg = acc + 1.0
        iv = plsc.bitcast(deg, jnp.int32)
        y = plsc.bitcast(jnp.int32(0x5F3759DF) - (iv >> 1), jnp.float32)
        for _ in range(3):
            y = y * (1.5 - 0.5 * deg * y * y)
        disv[pl.ds(k * L, L)] = y
        return 0
    lax.fori_loop(0, COLS // L, red, 0)

    @pl.when((c == 0) & (s < NS - 1))
    def _():
        pltpu.sync_copy(disv, dis_hbm.at[pl.ds(s * COLS, COLS)])

    TAIL = N - (NS - 1) * COLS
    @pl.when((c == 0) & (s == NS - 1))
    def _():
        pltpu.sync_copy(disv.at[pl.ds(0, TAIL)],
                        dis_hbm.at[pl.ds((NS - 1) * COLS, TAIL)])


def _deg_kernel(dst):
    k = pl.kernel(
        _deg_body,
        out_type=jax.ShapeDtypeStruct((N,), jnp.float32),
        mesh=_mesh(),
        compiler_params=pltpu.CompilerParams(needs_layout_passes=False),
        scratch_types=[
            pltpu.VMEM((DEG_EPT,), jnp.int32),
            pltpu.VMEM((NP,), jnp.float32),
            pltpu.VMEM((NS, COLS), jnp.float32),
            pltpu.VMEM((COLS,), jnp.float32),
            pltpu.VMEM_SHARED((NS, NP), jnp.float32),
            pltpu.SemaphoreType.DMA,
        ],
    )
    return k(dst)


def _mm_body(w_ref, x_ref, dis_ref, hpt_ref):
    h = lax.dot_general(w_ref[...], x_ref[...], (((0,), (1,)), ((), ())),
                        preferred_element_type=jnp.float32)
    hpt_ref[...] = h * dis_ref[...]


def _mm_kernel(W, x, disT):
    return pl.pallas_call(
        _mm_body,
        out_shape=jax.ShapeDtypeStruct((D, N), jnp.float32),
    )(W, x, disT)


def _agg_body(hpt_hbm, src_hbm, dst_hbm, aggt_hbm,
              sb0, db0, sb1, db1, p0, p1, p2, p3, a0, a1, a2, a3,
              sem0, sem1, semp, semw):
    c = lax.axis_index("c")
    s = lax.axis_index("s")
    wid = c * NS + s
    zero16 = jnp.zeros((L,), jnp.float32)
    panels = [p0, p1, p2, p3]
    aggs = [a0, a1, a2, a3]

    def start_stage(jb, sb, db, sem):
        pltpu.make_async_copy(src_hbm.at[pl.ds(jb * EB, EB)],
                              sb.at[pl.ds(0, EB)], sem).start()
        pltpu.make_async_copy(dst_hbm.at[pl.ds(jb * EB, EB)],
                              db.at[pl.ds(0, EB)], sem).start()

    def wait_stage(sb, db, sem):
        pltpu.make_async_copy(src_hbm.at[pl.ds(0, EB)],
                              sb.at[pl.ds(0, EB)], sem).wait()
        pltpu.make_async_copy(dst_hbm.at[pl.ds(0, EB)],
                              db.at[pl.ds(0, EB)], sem).wait()

    def compute(sb, db):
        n = EB // L
        s0 = sb[pl.ds(0, L)]
        d0 = db[pl.ds(0, L)]
        s1 = sb[pl.ds(L, L)]
        d1 = db[pl.ds(L, L)]
        v0 = tuple(plsc.load_gather(panels[k], [s0]) for k in range(PW))

        def body(i, carry):
            s_cur, d_cur, d_prev = carry[0], carry[1], carry[2]
            vs = carry[3:]
            s_next = sb[pl.ds((i + 1) * L, L)]
            d_next = db[pl.ds((i + 1) * L, L)]
            v_cur = tuple(plsc.load_gather(panels[k], [s_cur])
                          for k in range(PW))
            for k in range(PW):
                plsc.addupdate_scatter(aggs[k], [d_prev], vs[k])
            return (s_next, d_next, d_cur) + v_cur

        carry = lax.fori_loop(1, n, body, (s1, d1, d0) + v0)
        d_last, v_last = carry[2], carry[3:]
        for k in range(PW):
            plsc.addupdate_scatter(aggs[k], [d_last], v_last[k])

    for p in range(NPASS):
        cb = p * (NC * NS * PW) + wid * PW

        for r in range(PW):
            pltpu.make_async_copy(hpt_hbm.at[cb + r], panels[r], semp).start()
        if p > 0:
            pcb = (p - 1) * (NC * NS * PW) + wid * PW
            for r in range(PW):
                pltpu.make_async_copy(aggs[r], aggt_hbm.at[pcb + r],
                                      semw).wait()
        start_stage(0, sb0, db0, sem0)

        def zloop(i, _):
            for k in range(PW):
                aggs[k][pl.ds(i * L, L)] = zero16
            return 0
        lax.fori_loop(0, N // L, zloop, 0)

        for r in range(PW):
            pltpu.make_async_copy(hpt_hbm.at[cb + r], panels[r], semp).wait()

        def outer(g, _):
            j1 = 2 * g + 1
            wait_stage(sb0, db0, sem0)
            start_stage(j1, sb1, db1, sem1)
            compute(sb0, db0)
            wait_stage(sb1, db1, sem1)

            @pl.when(j1 + 1 < NCH)
            def _():
                start_stage(j1 + 1, sb0, db0, sem0)
            compute(sb1, db1)
            return 0
        lax.fori_loop(0, NCH // 2, outer, 0)

        for r in range(PW):
            pltpu.make_async_copy(aggs[r], aggt_hbm.at[cb + r], semw).start()
        if p == NPASS - 1:
            for r in range(PW):
                pltpu.make_async_copy(aggs[r], aggt_hbm.at[cb + r],
                                      semw).wait()


def _agg_kernel(hpT, src, dst):
    k = pl.kernel(
        _agg_body,
        out_type=jax.ShapeDtypeStruct((D, N), jnp.float32),
        mesh=_mesh(),
        compiler_params=pltpu.CompilerParams(needs_layout_passes=False),
        scratch_types=[
            pltpu.VMEM((EB + 2 * L,), jnp.int32),
            pltpu.VMEM((EB + 2 * L,), jnp.int32),
            pltpu.VMEM((EB + 2 * L,), jnp.int32),
            pltpu.VMEM((EB + 2 * L,), jnp.int32),
            pltpu.VMEM((N,), jnp.float32),
            pltpu.VMEM((N,), jnp.float32),
            pltpu.VMEM((N,), jnp.float32),
            pltpu.VMEM((N,), jnp.float32),
            pltpu.VMEM((N,), jnp.float32),
            pltpu.VMEM((N,), jnp.float32),
            pltpu.VMEM((N,), jnp.float32),
            pltpu.VMEM((N,), jnp.float32),
            pltpu.SemaphoreType.DMA,
            pltpu.SemaphoreType.DMA,
            pltpu.SemaphoreType.DMA,
            pltpu.SemaphoreType.DMA,
        ],
    )
    return k(hpT, src, dst)


def _ln_body(agg_ref, hp_ref, dis_ref, x_ref, b_ref, g_ref, be_ref, o_ref):
    pre = dis_ref[...] * (agg_ref[...] + hp_ref[...]) + b_ref[...]
    mu = jnp.mean(pre, axis=0, keepdims=True)
    var = jnp.mean((pre - mu) ** 2, axis=0, keepdims=True)
    y = (pre - mu) * lax.rsqrt(var + 1e-5) * g_ref[...] + be_ref[...]
    o_ref[...] = jnp.maximum(y.T + x_ref[...], 0.0)


def _ln_kernel(aggT, hpT, disT, x, b, gamma, beta):
    mspec = pl.BlockSpec((D, BN), lambda i: (0, i))
    nspec = pl.BlockSpec((BN, D), lambda i: (i, 0))
    vspec = pl.BlockSpec((D, 1), lambda i: (0, 0))
    return pl.pallas_call(
        _ln_body,
        grid=(pl.cdiv(N, BN),),
        in_specs=[mspec, mspec, pl.BlockSpec((1, BN), lambda i: (0, i)),
                  nspec, vspec, vspec, vspec],
        out_specs=nspec,
        out_shape=jax.ShapeDtypeStruct((N, D), jnp.float32),
    )(aggT, hpT, disT, x, b.reshape(D, 1), gamma.reshape(D, 1),
      beta.reshape(D, 1))


def kernel(x, edge_index, W, b, gamma, beta):
    src = edge_index[0]
    dst = edge_index[1]
    dis = _deg_kernel(dst)
    disT = dis.reshape(1, N)
    hpT = _mm_kernel(W, x, disT)
    aggT = _agg_kernel(hpT, src, dst)
    return _ln_kernel(aggT, hpT, disT, x, b, gamma, beta)


# speedup vs baseline: 1.0157x; 1.0157x over previous
import jax
import jax.numpy as jnp
from jax import lax
from jax.experimental import pallas as pl
from jax.experimental.pallas import tpu as pltpu
from jax.experimental.pallas import tpu_sc as plsc

N = 10000
E = 160000
D = 256

NC = 2
NS = 16
L = 16

NP = 10240
COLS = NP // NS
DEG_EPT = E // NS

PW = 4
NPASS = D // (NC * NS * PW)
EB = 3200
NCH = E // EB

BN = 1280


def _mesh():
    return plsc.VectorSubcoreMesh(core_axis_name="c", subcore_axis_name="s",
                                  num_cores=NC, num_subcores=NS)


def _deg_body(dst_hbm, dis_hbm, dstv, degv, colbuf, disv, spm, semr):
    c = lax.axis_index("c")
    s = lax.axis_index("s")
    zero16 = jnp.zeros((L,), jnp.float32)
    ones16 = jnp.ones((L,), jnp.float32)

    def zloop(i, _):
        degv[pl.ds(i * L, L)] = zero16
        return 0
    lax.fori_loop(0, NP // L, zloop, 0)

    pltpu.sync_copy(dst_hbm.at[pl.ds(s * DEG_EPT, DEG_EPT)], dstv)

    def hist(i, _):
        d = dstv[pl.ds(i * L, L)]
        plsc.addupdate_scatter(degv, [d], ones16)
        return 0
    lax.fori_loop(0, DEG_EPT // L, hist, 0)

    pltpu.sync_copy(degv, spm.at[s])
    plsc.subcore_barrier()

    for r in range(NS):
        pltpu.make_async_copy(spm.at[r, pl.ds(s * COLS, COLS)],
                              colbuf.at[r], semr).start()
    for r in range(NS):
        pltpu.make_async_copy(spm.at[r, pl.ds(s * COLS, COLS)],
                              colbuf.at[r], semr).wait()

    def red(k, _):
        acc = colbuf[0, pl.ds(k * L, L)]
        for r in range(1, NS):
            acc = acc + colbuf[r, pl.ds(k * L, L)]
        deg = acc + 1.0
        iv = plsc.bitcast(deg, jnp.int32)
        y = plsc.bitcast(jnp.int32(0x5F3759DF) - (iv >> 1), jnp.float32)
        for _ in range(3):
            y = y * (1.5 - 0.5 * deg * y * y)
        disv[pl.ds(k * L, L)] = y
        return 0
    lax.fori_loop(0, COLS // L, red, 0)

    @pl.when((c == 0) & (s < NS - 1))
    def _():
        pltpu.sync_copy(disv, dis_hbm.at[pl.ds(s * COLS, COLS)])

    TAIL = N - (NS - 1) * COLS
    @pl.when((c == 0) & (s == NS - 1))
    def _():
        pltpu.sync_copy(disv.at[pl.ds(0, TAIL)],
                        dis_hbm.at[pl.ds((NS - 1) * COLS, TAIL)])


def _deg_kernel(dst):
    k = pl.kernel(
        _deg_body,
        out_type=jax.ShapeDtypeStruct((N,), jnp.float32),
        mesh=_mesh(),
        compiler_params=pltpu.CompilerParams(needs_layout_passes=False),
        scratch_types=[
            pltpu.VMEM((DEG_EPT,), jnp.int32),
            pltpu.VMEM((NP,), jnp.float32),
            pltpu.VMEM((NS, COLS), jnp.float32),
            pltpu.VMEM((COLS,), jnp.float32),
            pltpu.VMEM_SHARED((NS, NP), jnp.float32),
            pltpu.SemaphoreType.DMA,
        ],
    )
    return k(dst)


def _mm_body(w_ref, x_ref, dis_ref, hpt_ref):
    h = lax.dot_general(w_ref[...], x_ref[...], (((0,), (1,)), ((), ())),
                        preferred_element_type=jnp.float32)
    hpt_ref[...] = h * dis_ref[...]


def _mm_kernel(W, x, disT):
    return pl.pallas_call(
        _mm_body,
        out_shape=jax.ShapeDtypeStruct((D, N), jnp.float32),
    )(W, x, disT)


def _agg_body(hpt_hbm, src_hbm, dst_hbm, aggt_hbm,
              sb0, db0, sb1, db1, p0, p1, p2, p3, a0, a1, a2, a3,
              sem0, sem1, semp, semw):
    c = lax.axis_index("c")
    s = lax.axis_index("s")
    wid = c * NS + s
    zero16 = jnp.zeros((L,), jnp.float32)
    panels = [p0, p1, p2, p3]
    aggs = [a0, a1, a2, a3]

    def start_stage(jb, sb, db, sem):
        pltpu.make_async_copy(src_hbm.at[pl.ds(jb * EB, EB)],
                              sb.at[pl.ds(0, EB)], sem).start()
        pltpu.make_async_copy(dst_hbm.at[pl.ds(jb * EB, EB)],
                              db.at[pl.ds(0, EB)], sem).start()

    def wait_stage(sb, db, sem):
        pltpu.make_async_copy(src_hbm.at[pl.ds(0, EB)],
                              sb.at[pl.ds(0, EB)], sem).wait()
        pltpu.make_async_copy(dst_hbm.at[pl.ds(0, EB)],
                              db.at[pl.ds(0, EB)], sem).wait()

    def compute(sb, db):
        n = EB // L
        s0 = sb[pl.ds(0, L)]
        d0 = db[pl.ds(0, L)]
        s1 = sb[pl.ds(L, L)]
        d1 = db[pl.ds(L, L)]
        v0 = tuple(plsc.load_gather(panels[k], [s0]) for k in range(PW))

        def body(i, carry):
            s_cur, d_cur, d_prev = carry[0], carry[1], carry[2]
            vs = carry[3:]
            s_next = sb[pl.ds((i + 1) * L, L)]
            d_next = db[pl.ds((i + 1) * L, L)]
            v_cur = tuple(plsc.load_gather(panels[k], [s_cur])
                          for k in range(PW))
            for k in range(PW):
                plsc.addupdate_scatter(aggs[k], [d_prev], vs[k])
            return (s_next, d_next, d_cur) + v_cur

        carry = lax.fori_loop(1, n, body, (s1, d1, d0) + v0, unroll=2)
        d_last, v_last = carry[2], carry[3:]
        for k in range(PW):
            plsc.addupdate_scatter(aggs[k], [d_last], v_last[k])

    for p in range(NPASS):
        cb = p * (NC * NS * PW) + wid * PW

        for r in range(PW):
            pltpu.make_async_copy(hpt_hbm.at[cb + r], panels[r], semp).start()
        if p > 0:
            pcb = (p - 1) * (NC * NS * PW) + wid * PW
            for r in range(PW):
                pltpu.make_async_copy(aggs[r], aggt_hbm.at[pcb + r],
                                      semw).wait()
        start_stage(0, sb0, db0, sem0)

        def zloop(i, _):
            for k in range(PW):
                aggs[k][pl.ds(i * L, L)] = zero16
            return 0
        lax.fori_loop(0, N // L, zloop, 0)

        for r in range(PW):
            pltpu.make_async_copy(hpt_hbm.at[cb + r], panels[r], semp).wait()

        def outer(g, _):
            j1 = 2 * g + 1
            wait_stage(sb0, db0, sem0)
            start_stage(j1, sb1, db1, sem1)
            compute(sb0, db0)
            wait_stage(sb1, db1, sem1)

            @pl.when(j1 + 1 < NCH)
            def _():
                start_stage(j1 + 1, sb0, db0, sem0)
            compute(sb1, db1)
            return 0
        lax.fori_loop(0, NCH // 2, outer, 0)

        for r in range(PW):
            pltpu.make_async_copy(aggs[r], aggt_hbm.at[cb + r], semw).start()
        if p == NPASS - 1:
            for r in range(PW):
                pltpu.make_async_copy(aggs[r], aggt_hbm.at[cb + r],
                                      semw).wait()


def _agg_kernel(hpT, src, dst):
    k = pl.kernel(
        _agg_body,
        out_type=jax.ShapeDtypeStruct((D, N), jnp.float32),
        mesh=_mesh(),
        compiler_params=pltpu.CompilerParams(needs_layout_passes=False),
        scratch_types=[
            pltpu.VMEM((EB + 2 * L,), jnp.int32),
            pltpu.VMEM((EB + 2 * L,), jnp.int32),
            pltpu.VMEM((EB + 2 * L,), jnp.int32),
            pltpu.VMEM((EB + 2 * L,), jnp.int32),
            pltpu.VMEM((N,), jnp.float32),
            pltpu.VMEM((N,), jnp.float32),
            pltpu.VMEM((N,), jnp.float32),
            pltpu.VMEM((N,), jnp.float32),
            pltpu.VMEM((N,), jnp.float32),
            pltpu.VMEM((N,), jnp.float32),
            pltpu.VMEM((N,), jnp.float32),
            pltpu.VMEM((N,), jnp.float32),
            pltpu.SemaphoreType.DMA,
            pltpu.SemaphoreType.DMA,
            pltpu.SemaphoreType.DMA,
            pltpu.SemaphoreType.DMA,
        ],
    )
    return k(hpT, src, dst)


def _ln_body(agg_ref, hp_ref, dis_ref, x_ref, b_ref, g_ref, be_ref, o_ref):
    pre = dis_ref[...] * (agg_ref[...] + hp_ref[...]) + b_ref[...]
    mu = jnp.mean(pre, axis=0, keepdims=True)
    var = jnp.mean((pre - mu) ** 2, axis=0, keepdims=True)
    y = (pre - mu) * lax.rsqrt(var + 1e-5) * g_ref[...] + be_ref[...]
    o_ref[...] = jnp.maximum(y.T + x_ref[...], 0.0)


def _ln_kernel(aggT, hpT, disT, x, b, gamma, beta):
    mspec = pl.BlockSpec((D, BN), lambda i: (0, i))
    nspec = pl.BlockSpec((BN, D), lambda i: (i, 0))
    vspec = pl.BlockSpec((D, 1), lambda i: (0, 0))
    return pl.pallas_call(
        _ln_body,
        grid=(pl.cdiv(N, BN),),
        in_specs=[mspec, mspec, pl.BlockSpec((1, BN), lambda i: (0, i)),
                  nspec, vspec, vspec, vspec],
        out_specs=nspec,
        out_shape=jax.ShapeDtypeStruct((N, D), jnp.float32),
    )(aggT, hpT, disT, x, b.reshape(D, 1), gamma.reshape(D, 1),
      beta.reshape(D, 1))


def kernel(x, edge_index, W, b, gamma, beta):
    src = edge_index[0]
    dst = edge_index[1]
    dis = _deg_kernel(dst)
    disT = dis.reshape(1, N)
    hpT = _mm_kernel(W, x, disT)
    aggT = _agg_kernel(hpT, src, dst)
    return _ln_kernel(aggT, hpT, disT, x, b, gamma, beta)
